# baseline (speedup 1.0000x reference)
import functools

import jax
import jax.numpy as jnp
from jax import lax
from jax.experimental import pallas as pl
from jax.experimental.pallas import tpu as pltpu
from jax.experimental.pallas import tpu_sc as plsc

N_CAT = 26
N_NUM = 13
VOCAB = 100000
EMB = 32
B = 4096
D_EMB = N_CAT * EMB
PACKED_ROWS = VOCAB // 4
NBLK_FULL = N_CAT * 781
R_TOTAL = B * N_CAT


def _iota16():
    return lax.iota(jnp.int32, 16)


@functools.cache
def _make_sc_detranspose():
    info = plsc.get_sparse_core_info()
    nw = info.num_cores * info.num_subcores
    iters = 636
    mesh = plsc.VectorSubcoreMesh(core_axis_name="c", subcore_axis_name="s")

    @functools.partial(
        pl.kernel,
        mesh=mesh,
        out_type=jax.ShapeDtypeStruct((N_CAT, PACKED_ROWS, 128), jnp.float32),
        scratch_types=[
            pltpu.VMEM((2, 32, 129), jnp.float32),
            pltpu.VMEM((2, 32, 128), jnp.float32),
            pltpu.VMEM((32, 33), jnp.float32),
            pltpu.SemaphoreType.DMA,
            pltpu.SemaphoreType.DMA,
            pltpu.SemaphoreType.DMA,
            pltpu.SemaphoreType.DMA,
        ],
        compiler_params=pltpu.CompilerParams(needs_layout_passes=False),
    )
    def detrans_k(tt_hbm, packed_hbm, in_b, st_b, tail_b, is0, is1, os0, os1):
        wid = lax.axis_index("s") * info.num_cores + lax.axis_index("c")
        isems = (is0, is1)
        osems = (os0, os1)
        r0 = _iota16()
        r1 = r0 + 16
        def transpose_block(src_ref, dst_ref, nrows):
            pending = []
            for pp in range(nrows):
                for s in range(8):
                    rows = r0 if s % 2 == 0 else r1
                    col = jnp.full((16,), 4 * pp + s // 2, jnp.int32)
                    v = plsc.load_gather(src_ref, [rows, col])
                    pending.append((pp, s, v))
                    if len(pending) >= 24:
                        ppo, so, vo = pending.pop(0)
                        dst_ref[ppo, pl.ds(16 * so, 16)] = vo
            for ppo, so, vo in pending:
                dst_ref[ppo, pl.ds(16 * so, 16)] = vo

        def fj(k):
            bid = jnp.minimum(wid + nw * k, NBLK_FULL - 1)
            f = bid // 781
            return f, bid - f * 781

        def in_copies(k, p):
            f, j = fj(k)
            return [
                pltpu.make_async_copy(
                    tt_hbm.at[f, e, pl.ds(j * 128, 128)],
                    in_b.at[p, e, pl.ds(0, 128)],
                    isems[p],
                )
                for e in range(32)
            ]

        def start_in(k, p):
            for cp in in_copies(k, p):
                cp.start()

        start_in(jnp.int32(0), 0)
        start_in(jnp.int32(1), 1)

        def body(i, carry):
            k0 = 2 * i
            for p in range(2):
                k = k0 + p
                f, j = fj(k)
                for cp in in_copies(k, p):
                    cp.wait()

                @pl.when(k0 >= 2)
                def _():
                    pltpu.make_async_copy(
                        st_b.at[p],
                        packed_hbm.at[f, pl.ds(32 * j, 32), :],
                        osems[p],
                    ).wait()

                transpose_block(in_b.at[p], st_b.at[p], 32)
                pltpu.make_async_copy(
                    st_b.at[p], packed_hbm.at[f, pl.ds(32 * j, 32), :], osems[p]
                ).start()
                start_in(k + 2, p)
            return carry

        lax.fori_loop(0, iters // 2, body, jnp.int32(0))

        for p in range(2):
            for cp in in_copies(jnp.int32(iters + p), p):
                cp.wait()
            fl, jl = fj(jnp.int32(iters - 2 + p))
            pltpu.make_async_copy(
                st_b.at[p], packed_hbm.at[fl, pl.ds(32 * jl, 32), :], osems[p]
            ).wait()

        @pl.when(wid < N_CAT)
        def _():
            f = wid
            for e in range(32):
                pltpu.sync_copy(
                    tt_hbm.at[f, e, pl.ds(99968, 32)],
                    tail_b.at[e, pl.ds(0, 32)],
                )
            transpose_block(tail_b, st_b.at[0], 8)
            pltpu.sync_copy(
                st_b.at[0, pl.ds(0, 8), :],
                packed_hbm.at[f, pl.ds(24992, 8), :],
            )

    return detrans_k


@functools.cache
def _make_sc_lookup():
    info = plsc.get_sparse_core_info()
    nw = info.num_cores * info.num_subcores
    rpw = R_TOTAL // nw
    ck = 416
    nck = rpw // ck
    mesh = plsc.VectorSubcoreMesh(core_axis_name="c", subcore_axis_name="s")

    @functools.partial(
        pl.kernel,
        mesh=mesh,
        out_type=jax.ShapeDtypeStruct((R_TOTAL // 4, 128), jnp.float32),
        scratch_types=[
            pltpu.VMEM((rpw,), jnp.int32),
            pltpu.VMEM((rpw,), jnp.int32),
            pltpu.VMEM((rpw,), jnp.int32),
            pltpu.VMEM((ck, 128), jnp.float32),
            pltpu.VMEM((ck // 4, 128), jnp.float32),
            pltpu.SemaphoreType.DMA,
        ],
        compiler_params=pltpu.CompilerParams(needs_layout_passes=False),
    )
    def lookup_k(pos_hbm, packed_hbm, out_hbm, pos_v, idx4_v, off_v, gbuf, obuf, sem):
        wid = lax.axis_index("s") * info.num_cores + lax.axis_index("c")
        base = wid * rpw
        pltpu.sync_copy(pos_hbm.at[pl.ds(base, rpw)], pos_v)

        def idx_body(i, carry):
            p16 = pos_v[pl.ds(16 * i, 16)]
            idx4_v[pl.ds(16 * i, 16)] = lax.shift_right_logical(p16, 2)
            off_v[pl.ds(16 * i, 16)] = lax.shift_left(
                jnp.bitwise_and(p16, 3), 5
            )
            return carry

        lax.fori_loop(0, rpw // 16, idx_body, jnp.int32(0))

        qm = _iota16()
        for c in range(nck):
            pltpu.async_copy(
                packed_hbm.at[idx4_v.at[pl.ds(c * ck, ck)]], gbuf, sem
            ).wait()

            def sel_body(i, carry):
                L0 = 8 * i
                lrows = [jnp.full((16,), L0 + t + c * ck, jnp.int32) for t in range(8)]
                offs = [plsc.load_gather(off_v, [lrows[t]]) for t in range(8)]
                cols = [offs[t] + qm for t in range(8)]
                grows = [jnp.full((16,), L0 + t, jnp.int32) for t in range(8)]
                v0s = [plsc.load_gather(gbuf, [grows[t], cols[t]]) for t in range(8)]
                v1s = [plsc.load_gather(gbuf, [grows[t], cols[t] + 16]) for t in range(8)]
                orow0 = 2 * i
                for t in range(8):
                    orow = orow0 + t // 4
                    ocol = 32 * (t % 4)
                    obuf[orow, pl.ds(ocol, 16)] = v0s[t]
                    obuf[orow, pl.ds(ocol + 16, 16)] = v1s[t]
                return carry

            lax.fori_loop(0, ck // 8, sel_body, jnp.int32(0))
            pltpu.sync_copy(
                obuf, out_hbm.at[pl.ds(wid * (rpw // 4) + c * (ck // 4), ck // 4)]
            )

    return lookup_k


def _mlp_body(emb, xn, w1a, w1b, b1r, w2, b2r, w3, b3r, out):
    h = jnp.dot(emb[...], w1a[...], preferred_element_type=jnp.float32)
    h = h + jnp.dot(xn[...], w1b[...], preferred_element_type=jnp.float32)
    h = jnp.maximum(h + b1r[...], 0.0)
    h = jnp.dot(h, w2[...], preferred_element_type=jnp.float32) + b2r[...]
    h = jnp.maximum(h, 0.0)
    h = jnp.dot(h, w3[...], preferred_element_type=jnp.float32) + b3r[...]
    out[...] = jnp.maximum(h, 0.0)


@functools.cache
def _make_mlp(tb: int):
    grid = (B // tb,)
    return pl.pallas_call(
        _mlp_body,
        grid=grid,
        in_specs=[
            pl.BlockSpec((tb, D_EMB), lambda i: (i, 0)),
            pl.BlockSpec((tb, N_NUM), lambda i: (i, 0)),
            pl.BlockSpec((D_EMB, 512), lambda i: (0, 0)),
            pl.BlockSpec((N_NUM, 512), lambda i: (0, 0)),
            pl.BlockSpec((1, 512), lambda i: (0, 0)),
            pl.BlockSpec((512, 256), lambda i: (0, 0)),
            pl.BlockSpec((1, 256), lambda i: (0, 0)),
            pl.BlockSpec((256, 128), lambda i: (0, 0)),
            pl.BlockSpec((1, 128), lambda i: (0, 0)),
        ],
        out_specs=pl.BlockSpec((tb, 128), lambda i: (i, 0)),
        out_shape=jax.ShapeDtypeStruct((B, 128), jnp.float32),
    )


def kernel(x, tables, W1, b1, W2, b2, W3, b3):
    idx = x[:, :N_CAT].astype(jnp.int32)
    offsets = (jnp.arange(N_CAT, dtype=jnp.int32) * VOCAB)[None, :]
    pos = (idx + offsets).reshape(-1)
    x_num = x[:, N_CAT:]
    tt = tables.transpose(0, 2, 1)
    packed = _make_sc_detranspose()(tt)
    packed2d = packed.reshape(N_CAT * PACKED_ROWS, 128)
    emb_packed = _make_sc_lookup()(pos, packed2d)
    emb = emb_packed.reshape(B, D_EMB)

    out = _make_mlp(512)(
        emb,
        x_num,
        W1[:D_EMB],
        W1[D_EMB:],
        b1[None, :],
        W2,
        b2[None, :],
        W3,
        b3[None, :],
    )
    return out


# speedup vs baseline: 1.5455x; 1.5455x over previous
import functools

import jax
import jax.numpy as jnp
from jax import lax
from jax.experimental import pallas as pl
from jax.experimental.pallas import tpu as pltpu
from jax.experimental.pallas import tpu_sc as plsc

N_CAT = 26
N_NUM = 13
VOCAB = 100000
EMB = 32
B = 4096
D_EMB = N_CAT * EMB


@functools.cache
def _make_sc_gather(n_rows: int, d: int):
    info = plsc.get_sparse_core_info()
    nw = info.num_cores * info.num_subcores
    assert n_rows % (8 * nw) == 0
    r_per_w = n_rows // nw
    mesh = plsc.VectorSubcoreMesh(core_axis_name="c", subcore_axis_name="s")

    @functools.partial(
        pl.kernel,
        mesh=mesh,
        out_type=jax.ShapeDtypeStruct((n_rows, d), jnp.float32),
        scratch_types=[
            pltpu.VMEM((r_per_w,), jnp.int32),
            pltpu.VMEM((r_per_w, d), jnp.float32),
            pltpu.SemaphoreType.DMA,
        ],
        compiler_params=pltpu.CompilerParams(use_tc_tiling_on_sc=False),
    )
    def gather_k(idx_hbm, table_hbm, out_hbm, idx_v, rows_v, sem):
        wid = lax.axis_index("s") * info.num_cores + lax.axis_index("c")
        base = wid * r_per_w
        pltpu.sync_copy(idx_hbm.at[pl.ds(base, r_per_w)], idx_v)
        pltpu.async_copy(table_hbm.at[idx_v], rows_v, sem).wait()
        pltpu.sync_copy(rows_v, out_hbm.at[pl.ds(base, r_per_w)])

    return gather_k


def _mlp_body(emb, xn, w1a, w1b, b1r, w2, b2r, w3, b3r, out):
    h = jnp.dot(emb[...], w1a[...], preferred_element_type=jnp.float32)
    h = h + jnp.dot(xn[...], w1b[...], preferred_element_type=jnp.float32)
    h = jnp.maximum(h + b1r[...], 0.0)
    h = jnp.dot(h, w2[...], preferred_element_type=jnp.float32) + b2r[...]
    h = jnp.maximum(h, 0.0)
    h = jnp.dot(h, w3[...], preferred_element_type=jnp.float32) + b3r[...]
    out[...] = jnp.maximum(h, 0.0)


@functools.cache
def _make_mlp(tb: int):
    grid = (B // tb,)
    return pl.pallas_call(
        _mlp_body,
        grid=grid,
        in_specs=[
            pl.BlockSpec((tb, D_EMB), lambda i: (i, 0)),
            pl.BlockSpec((tb, N_NUM), lambda i: (i, 0)),
            pl.BlockSpec((D_EMB, 512), lambda i: (0, 0)),
            pl.BlockSpec((N_NUM, 512), lambda i: (0, 0)),
            pl.BlockSpec((1, 512), lambda i: (0, 0)),
            pl.BlockSpec((512, 256), lambda i: (0, 0)),
            pl.BlockSpec((1, 256), lambda i: (0, 0)),
            pl.BlockSpec((256, 128), lambda i: (0, 0)),
            pl.BlockSpec((1, 128), lambda i: (0, 0)),
        ],
        out_specs=pl.BlockSpec((tb, 128), lambda i: (i, 0)),
        out_shape=jax.ShapeDtypeStruct((B, 128), jnp.float32),
    )


def kernel(x, tables, W1, b1, W2, b2, W3, b3):
    idx = x[:, :N_CAT].astype(jnp.int32)
    offsets = (jnp.arange(N_CAT, dtype=jnp.int32) * VOCAB)[None, :]
    flat_idx = (idx + offsets).reshape(-1)
    table2d = tables.reshape(N_CAT * VOCAB, EMB)
    x_num = x[:, N_CAT:]

    emb = _make_sc_gather(B * N_CAT, EMB)(flat_idx, table2d)
    emb = emb.reshape(B, D_EMB)

    out = _make_mlp(512)(
        emb,
        x_num,
        W1[:D_EMB],
        W1[D_EMB:],
        b1[None, :],
        W2,
        b2[None, :],
        W3,
        b3[None, :],
    )
    return out
